# baseline (speedup 1.0000x reference)
import jax
import jax.numpy as jnp
import numpy as np
from jax.experimental import pallas as pl
from jax.experimental.pallas import tpu as pltpu

T = 8192
D = 4096
E = 64
K = 8
BLK = 1024
GRID = T // BLK
DSPLIT = 2
DCHUNK = D // DSPLIT

_KS0 = 0
_KS1 = 12345
_KS2 = int(np.uint32(_KS0) ^ np.uint32(_KS1) ^ np.uint32(0x1BD11BDA))
_ROTS_A = (13, 15, 26, 6)
_ROTS_B = (17, 29, 16, 24)
_U32 = lambda v: jnp.int32(np.uint32(v).astype(np.int32))
_LO = np.nextafter(np.float32(-1.0), np.float32(0.0), dtype=np.float32)
_SCALE = np.float32(1.0) - _LO
_SQRT2 = np.float32(np.sqrt(2.0))


def _rotl(v, d):
    return jax.lax.shift_left(v, jnp.int32(d)) | jax.lax.shift_right_logical(
        v, jnp.int32(32 - d))


def _threefry_rounds(x0, x1, rots):
    for r in rots:
        x0 = x0 + x1
        x1 = _rotl(x1, r)
        x1 = x0 ^ x1
    return x0, x1


def _eps_block(i):
    n = (i * jnp.int32(BLK * E)
         + jax.lax.broadcasted_iota(jnp.int32, (BLK, E), 0) * jnp.int32(E)
         + jax.lax.broadcasted_iota(jnp.int32, (BLK, E), 1))
    x0 = jnp.zeros((BLK, E), jnp.int32) + _U32(_KS0)
    x1 = n + _U32(_KS1)
    x0, x1 = _threefry_rounds(x0, x1, _ROTS_A)
    x0, x1 = x0 + _U32(_KS1), x1 + _U32(_KS2 + 1)
    x0, x1 = _threefry_rounds(x0, x1, _ROTS_B)
    x0, x1 = x0 + _U32(_KS2), x1 + _U32(_KS0 + 2)
    x0, x1 = _threefry_rounds(x0, x1, _ROTS_A)
    x0, x1 = x0 + _U32(_KS0), x1 + _U32(_KS1 + 3)
    x0, x1 = _threefry_rounds(x0, x1, _ROTS_B)
    x0, x1 = x0 + _U32(_KS1), x1 + _U32(_KS2 + 4)
    x0, x1 = _threefry_rounds(x0, x1, _ROTS_A)
    x0, x1 = x0 + _U32(_KS2), x1 + _U32(_KS0 + 5)
    bits = x0 ^ x1
    fb = jax.lax.shift_right_logical(bits, jnp.int32(9)) | _U32(0x3F800000)
    f = jax.lax.bitcast_convert_type(fb, jnp.float32) - jnp.float32(1.0)
    u = jnp.maximum(jnp.float32(_LO), f * jnp.float32(_SCALE) + jnp.float32(_LO))
    return _SQRT2 * _erf_inv32(u)


_ERFINV_LT = (2.81022636e-08, 3.43273939e-07, -3.5233877e-06, -4.39150654e-06,
              0.00021858087, -0.00125372503, -0.00417768164, 0.246640727,
              1.50140941)
_ERFINV_GT = (-0.000200214257, 0.000100950558, 0.00134934322, -0.00367342844,
              0.00573950773, -0.0076224613, 0.00943887047, 1.00167406,
              2.83297682)


def _log1p32(t):
    u = t + jnp.float32(1.0)
    return jnp.where(u == jnp.float32(1.0), t,
                     t * jnp.log(u) / (u - jnp.float32(1.0)))


def _erf_inv32(x):
    w = -_log1p32(x * -x)
    lt = w < jnp.float32(5.0)
    w = jnp.where(lt, w - jnp.float32(2.5), jnp.sqrt(w) - jnp.float32(3.0))
    p = jnp.where(lt, jnp.float32(_ERFINV_LT[0]), jnp.float32(_ERFINV_GT[0]))
    for a, b in zip(_ERFINV_LT[1:], _ERFINV_GT[1:]):
        p = jnp.where(lt, jnp.float32(a), jnp.float32(b)) + p * w
    return jnp.where(jnp.abs(x) == jnp.float32(1.0), jnp.inf * x, p * x)


def _gate_kernel(x_ref, w_ref, gates_ref, idx_ref, aux_ref,
                 acc_ref, facc_ref, pacc_ref):
    i = pl.program_id(0)
    j = pl.program_id(1)
    part = jax.lax.dot_general(
        x_ref[...], w_ref[...], (((1,), (1,)), ((), ())),
        preferred_element_type=jnp.float32)

    @pl.when(j == 0)
    def _acc():
        acc_ref[...] = part

    @pl.when(j == DSPLIT - 1)
    def _epilogue():
        _finish(i, acc_ref[...] + part, gates_ref, idx_ref, aux_ref,
                facc_ref, pacc_ref)


def _finish(i, logits2, gates_ref, idx_ref, aux_ref, facc_ref, pacc_ref):
    clean = logits2[:, :E]
    nraw = logits2[:, E:]
    std = jax.nn.softplus(nraw)
    logits = clean + _eps_block(i) * std

    iota = jax.lax.broadcasted_iota(jnp.int32, (BLK, E), 1)
    u = jax.lax.bitcast_convert_type(logits, jnp.int32)
    code = jnp.where(u < 0, iota, E - 1 - iota)
    keys = jax.lax.bitcast_convert_type((u & ~jnp.int32(E - 1)) | code,
                                        jnp.float32)

    neg = jnp.float32(-jnp.inf)
    work = keys
    kmaxes = []
    for _ in range(K):
        m = jnp.max(work, axis=1, keepdims=True)
        work = jnp.where(work == m, neg, work)
        kmaxes.append(m)

    km = jnp.concatenate(kmaxes, axis=1)
    kb = jax.lax.bitcast_convert_type(km, jnp.int32)
    low = kb & jnp.int32(E - 1)
    idx_ref[...] = jnp.where(kb < 0, low, E - 1 - low)

    e = jnp.exp(logits - kmaxes[0])
    g = jnp.where(keys >= kmaxes[-1], e, 0.0)
    gates = g / jnp.sum(g, axis=1, keepdims=True)
    gates_ref[...] = gates
    p = e / jnp.sum(e, axis=1, keepdims=True)

    f_part = jnp.sum(gates, axis=0, keepdims=True)
    p_part = jnp.sum(p, axis=0, keepdims=True)

    @pl.when(i == 0)
    def _init():
        facc_ref[...] = jnp.zeros_like(facc_ref)
        pacc_ref[...] = jnp.zeros_like(pacc_ref)

    facc_ref[...] += f_part
    pacc_ref[...] += p_part

    @pl.when(i == GRID - 1)
    def _fin():
        s = (E / (T * T)) * jnp.sum(facc_ref[...] * pacc_ref[...],
                                    keepdims=True)
        aux_ref[...] = s.reshape(1, 1)


def kernel(x, w_gate, w_noise):
    w = jnp.concatenate([w_gate, w_noise], axis=0)
    gates, idx, aux = pl.pallas_call(
        _gate_kernel,
        grid=(GRID, DSPLIT),
        in_specs=[
            pl.BlockSpec((BLK, DCHUNK), lambda i, j: (i, j)),
            pl.BlockSpec((2 * E, DCHUNK), lambda i, j: (0, j)),
        ],
        out_specs=[
            pl.BlockSpec((BLK, E), lambda i, j: (i, 0)),
            pl.BlockSpec((BLK, K), lambda i, j: (i, 0)),
            pl.BlockSpec((1, 1), lambda i, j: (0, 0)),
        ],
        out_shape=[
            jax.ShapeDtypeStruct((T, E), jnp.float32),
            jax.ShapeDtypeStruct((T, K), jnp.int32),
            jax.ShapeDtypeStruct((1, 1), jnp.float32),
        ],
        scratch_shapes=[
            pltpu.VMEM((BLK, 2 * E), jnp.float32),
            pltpu.VMEM((1, E), jnp.float32),
            pltpu.VMEM((1, E), jnp.float32),
        ],
    )(x, w)
    return gates, idx, aux[0, 0]


# speedup vs baseline: 1.2787x; 1.2787x over previous
import jax
import jax.numpy as jnp
import numpy as np
from jax.experimental import pallas as pl
from jax.experimental.pallas import tpu as pltpu

T = 8192
D = 4096
E = 64
K = 8
BLK = 1024
GRID = T // BLK

_KS0 = 0
_KS1 = 12345
_KS2 = int(np.uint32(_KS0) ^ np.uint32(_KS1) ^ np.uint32(0x1BD11BDA))
_ROTS_A = (13, 15, 26, 6)
_ROTS_B = (17, 29, 16, 24)
_U32 = lambda v: jnp.int32(np.uint32(v).astype(np.int32))
_LO = np.nextafter(np.float32(-1.0), np.float32(0.0), dtype=np.float32)
_SCALE = np.float32(1.0) - _LO
_SQRT2 = np.float32(np.sqrt(2.0))


def _rotl(v, d):
    return jax.lax.shift_left(v, jnp.int32(d)) | jax.lax.shift_right_logical(
        v, jnp.int32(32 - d))


def _threefry_rounds(x0, x1, rots):
    for r in rots:
        x0 = x0 + x1
        x1 = _rotl(x1, r)
        x1 = x0 ^ x1
    return x0, x1


def _eps_block(i):
    n = (i * jnp.int32(BLK * E)
         + jax.lax.broadcasted_iota(jnp.int32, (BLK, E), 0) * jnp.int32(E)
         + jax.lax.broadcasted_iota(jnp.int32, (BLK, E), 1))
    x0 = jnp.zeros((BLK, E), jnp.int32) + _U32(_KS0)
    x1 = n + _U32(_KS1)
    x0, x1 = _threefry_rounds(x0, x1, _ROTS_A)
    x0, x1 = x0 + _U32(_KS1), x1 + _U32(_KS2 + 1)
    x0, x1 = _threefry_rounds(x0, x1, _ROTS_B)
    x0, x1 = x0 + _U32(_KS2), x1 + _U32(_KS0 + 2)
    x0, x1 = _threefry_rounds(x0, x1, _ROTS_A)
    x0, x1 = x0 + _U32(_KS0), x1 + _U32(_KS1 + 3)
    x0, x1 = _threefry_rounds(x0, x1, _ROTS_B)
    x0, x1 = x0 + _U32(_KS1), x1 + _U32(_KS2 + 4)
    x0, x1 = _threefry_rounds(x0, x1, _ROTS_A)
    x0, x1 = x0 + _U32(_KS2), x1 + _U32(_KS0 + 5)
    bits = x0 ^ x1
    fb = jax.lax.shift_right_logical(bits, jnp.int32(9)) | _U32(0x3F800000)
    f = jax.lax.bitcast_convert_type(fb, jnp.float32) - jnp.float32(1.0)
    u = jnp.maximum(jnp.float32(_LO), f * jnp.float32(_SCALE) + jnp.float32(_LO))
    return _SQRT2 * _erf_inv32(u)


_ERFINV_LT = (2.81022636e-08, 3.43273939e-07, -3.5233877e-06, -4.39150654e-06,
              0.00021858087, -0.00125372503, -0.00417768164, 0.246640727,
              1.50140941)
_ERFINV_GT = (-0.000200214257, 0.000100950558, 0.00134934322, -0.00367342844,
              0.00573950773, -0.0076224613, 0.00943887047, 1.00167406,
              2.83297682)


def _log1p32(t):
    u = t + jnp.float32(1.0)
    return jnp.where(u == jnp.float32(1.0), t,
                     t * jnp.log(u) / (u - jnp.float32(1.0)))


def _erf_inv32(x):
    w = -_log1p32(x * -x)
    lt = w < jnp.float32(5.0)
    w = jnp.where(lt, w - jnp.float32(2.5), jnp.sqrt(w) - jnp.float32(3.0))
    p = jnp.where(lt, jnp.float32(_ERFINV_LT[0]), jnp.float32(_ERFINV_GT[0]))
    for a, b in zip(_ERFINV_LT[1:], _ERFINV_GT[1:]):
        p = jnp.where(lt, jnp.float32(a), jnp.float32(b)) + p * w
    return jnp.where(jnp.abs(x) == jnp.float32(1.0), jnp.inf * x, p * x)


def _gate_kernel(x_ref, w_ref, gates_ref, idx_ref, aux_ref,
                 facc_ref, pacc_ref):
    i = pl.program_id(0)
    logits2 = jax.lax.dot_general(
        x_ref[...], w_ref[...], (((1,), (1,)), ((), ())),
        preferred_element_type=jnp.float32)
    clean = logits2[:, :E]
    nraw = logits2[:, E:]
    std = jax.nn.softplus(nraw)
    logits = clean + _eps_block(i) * std

    iota = jax.lax.broadcasted_iota(jnp.int32, (BLK, E), 1)
    u = jax.lax.bitcast_convert_type(logits, jnp.int32)
    code = jnp.where(u < 0, iota, E - 1 - iota)
    keys = jax.lax.bitcast_convert_type((u & ~jnp.int32(E - 1)) | code,
                                        jnp.float32)

    neg = jnp.float32(-jnp.inf)
    work = keys
    kmaxes = []
    for _ in range(K):
        m = jnp.max(work, axis=1, keepdims=True)
        work = jnp.where(work == m, neg, work)
        kmaxes.append(m)

    km = jnp.concatenate(kmaxes, axis=1)
    kb = jax.lax.bitcast_convert_type(km, jnp.int32)
    low = kb & jnp.int32(E - 1)
    idx_ref[...] = jnp.where(kb < 0, low, E - 1 - low)

    e = jnp.exp(logits - kmaxes[0])
    g = jnp.where(keys >= kmaxes[-1], e, 0.0)
    gates = g / jnp.sum(g, axis=1, keepdims=True)
    gates_ref[...] = gates
    p = e / jnp.sum(e, axis=1, keepdims=True)

    f_part = jnp.sum(gates, axis=0, keepdims=True)
    p_part = jnp.sum(p, axis=0, keepdims=True)

    @pl.when(i == 0)
    def _init():
        facc_ref[...] = jnp.zeros_like(facc_ref)
        pacc_ref[...] = jnp.zeros_like(pacc_ref)

    facc_ref[...] += f_part
    pacc_ref[...] += p_part

    @pl.when(i == GRID - 1)
    def _fin():
        s = (E / (T * T)) * jnp.sum(facc_ref[...] * pacc_ref[...],
                                    keepdims=True)
        aux_ref[...] = s.reshape(1, 1)


def kernel(x, w_gate, w_noise):
    w = jnp.concatenate([w_gate, w_noise], axis=0)
    gates, idx, aux = pl.pallas_call(
        _gate_kernel,
        grid=(GRID,),
        in_specs=[
            pl.BlockSpec((BLK, D), lambda i: (i, 0)),
            pl.BlockSpec((2 * E, D), lambda i: (0, 0)),
        ],
        out_specs=[
            pl.BlockSpec((BLK, E), lambda i: (i, 0)),
            pl.BlockSpec((BLK, K), lambda i: (i, 0)),
            pl.BlockSpec((1, 1), lambda i: (0, 0)),
        ],
        out_shape=[
            jax.ShapeDtypeStruct((T, E), jnp.float32),
            jax.ShapeDtypeStruct((T, K), jnp.int32),
            jax.ShapeDtypeStruct((1, 1), jnp.float32),
        ],
        scratch_shapes=[
            pltpu.VMEM((1, E), jnp.float32),
            pltpu.VMEM((1, E), jnp.float32),
        ],
    )(x, w)
    return gates, idx, aux[0, 0]


# speedup vs baseline: 1.2957x; 1.0133x over previous
import jax
import jax.numpy as jnp
import numpy as np
from jax.experimental import pallas as pl
from jax.experimental.pallas import tpu as pltpu

T = 8192
D = 4096
E = 64
K = 8
BLK = 1024
GRID = T // BLK

_KS0 = 0
_KS1 = 12345
_KS2 = int(np.uint32(_KS0) ^ np.uint32(_KS1) ^ np.uint32(0x1BD11BDA))
_ROTS_A = (13, 15, 26, 6)
_ROTS_B = (17, 29, 16, 24)
_U32 = lambda v: jnp.int32(np.uint32(v).astype(np.int32))
_LO = np.nextafter(np.float32(-1.0), np.float32(0.0), dtype=np.float32)
_SCALE = np.float32(1.0) - _LO
_SQRT2 = np.float32(np.sqrt(2.0))


def _rotl(v, d):
    return jax.lax.shift_left(v, jnp.int32(d)) | jax.lax.shift_right_logical(
        v, jnp.int32(32 - d))


def _threefry_rounds(x0, x1, rots):
    for r in rots:
        x0 = x0 + x1
        x1 = _rotl(x1, r)
        x1 = x0 ^ x1
    return x0, x1


def _eps_block(i):
    n = (i * jnp.int32(BLK * E)
         + jax.lax.broadcasted_iota(jnp.int32, (BLK, E), 0) * jnp.int32(E)
         + jax.lax.broadcasted_iota(jnp.int32, (BLK, E), 1))
    x0 = jnp.zeros((BLK, E), jnp.int32) + _U32(_KS0)
    x1 = n + _U32(_KS1)
    x0, x1 = _threefry_rounds(x0, x1, _ROTS_A)
    x0, x1 = x0 + _U32(_KS1), x1 + _U32(_KS2 + 1)
    x0, x1 = _threefry_rounds(x0, x1, _ROTS_B)
    x0, x1 = x0 + _U32(_KS2), x1 + _U32(_KS0 + 2)
    x0, x1 = _threefry_rounds(x0, x1, _ROTS_A)
    x0, x1 = x0 + _U32(_KS0), x1 + _U32(_KS1 + 3)
    x0, x1 = _threefry_rounds(x0, x1, _ROTS_B)
    x0, x1 = x0 + _U32(_KS1), x1 + _U32(_KS2 + 4)
    x0, x1 = _threefry_rounds(x0, x1, _ROTS_A)
    x0, x1 = x0 + _U32(_KS2), x1 + _U32(_KS0 + 5)
    bits = x0 ^ x1
    fb = jax.lax.shift_right_logical(bits, jnp.int32(9)) | _U32(0x3F800000)
    f = jax.lax.bitcast_convert_type(fb, jnp.float32) - jnp.float32(1.0)
    u = jnp.maximum(jnp.float32(_LO), f * jnp.float32(_SCALE) + jnp.float32(_LO))
    return _SQRT2 * _erf_inv32(u)


_ERFINV_LT = (2.81022636e-08, 3.43273939e-07, -3.5233877e-06, -4.39150654e-06,
              0.00021858087, -0.00125372503, -0.00417768164, 0.246640727,
              1.50140941)
_ERFINV_GT = (-0.000200214257, 0.000100950558, 0.00134934322, -0.00367342844,
              0.00573950773, -0.0076224613, 0.00943887047, 1.00167406,
              2.83297682)


def _log1p32(t):
    u = t + jnp.float32(1.0)
    return jnp.where(u == jnp.float32(1.0), t,
                     t * jnp.log(u) / (u - jnp.float32(1.0)))


def _erf_inv32(x):
    w = -_log1p32(x * -x)
    lt = w < jnp.float32(5.0)
    w = jnp.where(lt, w - jnp.float32(2.5), jnp.sqrt(w) - jnp.float32(3.0))
    p = jnp.where(lt, jnp.float32(_ERFINV_LT[0]), jnp.float32(_ERFINV_GT[0]))
    for a, b in zip(_ERFINV_LT[1:], _ERFINV_GT[1:]):
        p = jnp.where(lt, jnp.float32(a), jnp.float32(b)) + p * w
    return jnp.where(jnp.abs(x) == jnp.float32(1.0), jnp.inf * x, p * x)


def _gate_kernel(x_ref, w_ref, gates_ref, idx_ref, aux_ref,
                 facc_ref, pacc_ref):
    i = pl.program_id(0)
    logits2 = jax.lax.dot_general(
        x_ref[...], w_ref[...], (((1,), (1,)), ((), ())),
        preferred_element_type=jnp.float32)
    clean = logits2[:, :E]
    nraw = logits2[:, E:]
    std = jnp.maximum(nraw, jnp.float32(0.0)) + _log1p32(jnp.exp(-jnp.abs(nraw)))
    logits = clean + _eps_block(i) * std

    iota = jax.lax.broadcasted_iota(jnp.int32, (BLK, E), 1)
    u = jax.lax.bitcast_convert_type(logits, jnp.int32)
    code = jnp.where(u < 0, iota, E - 1 - iota)
    keys = jax.lax.bitcast_convert_type((u & ~jnp.int32(E - 1)) | code,
                                        jnp.float32)

    neg = jnp.float32(-jnp.inf)
    work = keys
    kmaxes = []
    for _ in range(K):
        m = jnp.max(work, axis=1, keepdims=True)
        work = jnp.where(work == m, neg, work)
        kmaxes.append(m)

    km = jnp.concatenate(kmaxes, axis=1)
    kb = jax.lax.bitcast_convert_type(km, jnp.int32)
    low = kb & jnp.int32(E - 1)
    idx_ref[...] = jnp.where(kb < 0, low, E - 1 - low)

    e = jnp.exp(logits - kmaxes[0])
    g = jnp.where(keys >= kmaxes[-1], e, 0.0)
    gates = g / jnp.sum(g, axis=1, keepdims=True)
    gates_ref[...] = gates
    p = e / jnp.sum(e, axis=1, keepdims=True)

    f_part = jnp.sum(gates, axis=0, keepdims=True)
    p_part = jnp.sum(p, axis=0, keepdims=True)

    @pl.when(i == 0)
    def _init():
        facc_ref[...] = jnp.zeros_like(facc_ref)
        pacc_ref[...] = jnp.zeros_like(pacc_ref)

    facc_ref[...] += f_part
    pacc_ref[...] += p_part

    @pl.when(i == GRID - 1)
    def _fin():
        s = (E / (T * T)) * jnp.sum(facc_ref[...] * pacc_ref[...],
                                    keepdims=True)
        aux_ref[...] = s.reshape(1, 1)


def kernel(x, w_gate, w_noise):
    w = jnp.concatenate([w_gate, w_noise], axis=0)
    gates, idx, aux = pl.pallas_call(
        _gate_kernel,
        grid=(GRID,),
        in_specs=[
            pl.BlockSpec((BLK, D), lambda i: (i, 0)),
            pl.BlockSpec((2 * E, D), lambda i: (0, 0)),
        ],
        out_specs=[
            pl.BlockSpec((BLK, E), lambda i: (i, 0)),
            pl.BlockSpec((BLK, K), lambda i: (i, 0)),
            pl.BlockSpec((1, 1), lambda i: (0, 0)),
        ],
        out_shape=[
            jax.ShapeDtypeStruct((T, E), jnp.float32),
            jax.ShapeDtypeStruct((T, K), jnp.int32),
            jax.ShapeDtypeStruct((1, 1), jnp.float32),
        ],
        scratch_shapes=[
            pltpu.VMEM((1, E), jnp.float32),
            pltpu.VMEM((1, E), jnp.float32),
        ],
    )(x, w)
    return gates, idx, aux[0, 0]
